# Initial kernel scaffold; baseline (speedup 1.0000x reference)
import jax
import jax.numpy as jnp
from jax.experimental import pallas as pl


def kernel(fv_features, points_img, proj_masks):
    raise NotImplementedError("write your pallas kernel here")


# speedup vs baseline: 1.0953x; 1.0953x over previous
import jax
import jax.numpy as jnp
from jax import lax
from jax.experimental import pallas as pl
from jax.experimental.pallas import tpu as pltpu
from jax.experimental.pallas import tpu_sc as plsc

_B, _C, _H, _W = 4, 32, 64, 2048
_N = _H * _W
_NX = _NY = 512
_NCELL = _NX * _NY
_NCHUNKS = 8
_CHUNK = _NCELL // _NCHUNKS
_DUMPS = 16
_ROWS = _CHUNK + _DUMPS

_NCORES = 2
_NSUB = 16
_PTS_PER_TILE = _N // _NSUB
_SB = 1024
_NSB = _PTS_PER_TILE // _SB
_G = 128
_NG = _SB // _G

_TROWS = _CHUNK // _NSUB
_ZR = 256

_XY0 = -51.2
_VOX = 0.2


def _sc_body(feats_hbm, xs_hbm, ys_hbm, ms_hbm, sums_hbm, cnts_hbm,
             feats_v, gidx_v, x_v, y_v, m_v, idx_v, ones_v, zrow_v, zcnt_v,
             sums_sh, cnts_sh):
  cid = lax.axis_index("c")
  tid = lax.axis_index("s")
  lanes = jnp.arange(16, dtype=jnp.int32)
  ones16 = jnp.ones((16,), jnp.float32)
  zeros16 = jnp.zeros((16,), jnp.float32)

  for g in range(_G // 16):
    ones_v[pl.ds(g * 16, 16)] = ones16

  def _zr(i, c):
    zrow_v[i, pl.ds(0, 16)] = zeros16
    zrow_v[i, pl.ds(16, 16)] = zeros16
    return c
  lax.fori_loop(0, _ZR, _zr, 0)

  def _zc(i, c):
    zcnt_v[pl.ds(i * 16, 16)] = zeros16
    return c
  lax.fori_loop(0, _SB // 16, _zc, 0)

  def _batch(b, carry0):
    for sb in range(_NSB):
      pbase = tid * _PTS_PER_TILE + sb * _SB
      pltpu.sync_copy(xs_hbm.at[b, pl.ds(pbase, _SB)], x_v)
      pltpu.sync_copy(ys_hbm.at[b, pl.ds(pbase, _SB)], y_v)
      pltpu.sync_copy(ms_hbm.at[b, pl.ds(pbase, _SB)], m_v)

      def _ci(k, c, sb=sb):
        o = k * 16
        xx = x_v[pl.ds(o, 16)]
        yy = y_v[pl.ds(o, 16)]
        mm = m_v[pl.ds(o, 16)]
        cx = ((xx - _XY0) / _VOX).astype(jnp.int32)
        cx = jnp.minimum(jnp.maximum(cx, 0), _NX - 1)
        cy = ((yy - _XY0) / _VOX).astype(jnp.int32)
        cy = jnp.minimum(jnp.maximum(cy, 0), _NY - 1)
        flat = cy * _NX + cx
        flat = jnp.where(mm > 0, flat, _NCELL)
        gidx_v[pl.ds(sb * _SB + o, 16)] = flat
        return c
      lax.fori_loop(0, _SB // 16, _ci, 0)

    def _pass(i, carry1):
      ch = i * _NCORES + cid
      cell0 = ch * _CHUNK

      for r in range(_TROWS // _ZR):
        pltpu.sync_copy(zrow_v, sums_sh.at[pl.ds(tid * _TROWS + r * _ZR, _ZR)])
      for r in range(_TROWS // _SB):
        pltpu.sync_copy(zcnt_v, cnts_sh.at[pl.ds(tid * _TROWS + r * _SB, _SB)])

      @pl.when(tid == 0)
      def _():
        pltpu.sync_copy(zrow_v.at[pl.ds(0, _DUMPS)],
                        sums_sh.at[pl.ds(_CHUNK, _DUMPS)])
        pltpu.sync_copy(zcnt_v.at[pl.ds(0, _DUMPS)],
                        cnts_sh.at[pl.ds(_CHUNK, _DUMPS)])

      plsc.subcore_barrier()

      for sb in range(_NSB):
        pbase = tid * _PTS_PER_TILE + sb * _SB
        pltpu.sync_copy(feats_hbm.at[b, pl.ds(pbase, _SB)], feats_v)

        for g in range(_NG):
          def _li(j, c, g=g, sb=sb):
            o = sb * _SB + g * _G + j * 16
            fl = gidx_v[pl.ds(o, 16)]
            loc = fl - cell0
            ok = (loc >= 0) & (loc < _CHUNK)
            idx_v[g, pl.ds(j * 16, 16)] = jnp.where(ok, loc, _CHUNK + lanes)
            return c
          lax.fori_loop(0, _G // 16, _li, 0)

        for g in range(_NG):
          pltpu.sync_copy(feats_v.at[pl.ds(g * _G, _G)],
                          sums_sh.at[idx_v.at[g]], add=True)
          pltpu.sync_copy(ones_v, cnts_sh.at[idx_v.at[g]], add=True)

      plsc.subcore_barrier()

      pltpu.sync_copy(sums_sh.at[pl.ds(tid * _TROWS, _TROWS)],
                      sums_hbm.at[b, pl.ds(cell0 + tid * _TROWS, _TROWS)])
      pltpu.sync_copy(cnts_sh.at[pl.ds(tid * _TROWS, _TROWS)],
                      cnts_hbm.at[b, pl.ds(cell0 + tid * _TROWS, _TROWS)])
      plsc.subcore_barrier()
      return carry1

    lax.fori_loop(0, _NCHUNKS // _NCORES, _pass, 0)
    return carry0

  lax.fori_loop(0, _B, _batch, 0)


def _sc_scatter(feats_t, xs, ys, ms):
  mesh = plsc.VectorSubcoreMesh(core_axis_name="c", subcore_axis_name="s",
                                num_cores=_NCORES, num_subcores=_NSUB)
  return pl.kernel(
      _sc_body,
      out_type=(
          jax.ShapeDtypeStruct((_B, _NCELL, _C), jnp.float32),
          jax.ShapeDtypeStruct((_B, _NCELL), jnp.float32),
      ),
      mesh=mesh,
      compiler_params=pltpu.CompilerParams(use_tc_tiling_on_sc=False),
      scratch_types=[
          pltpu.VMEM((_SB, _C), jnp.float32),
          pltpu.VMEM((_PTS_PER_TILE,), jnp.int32),
          pltpu.VMEM((_SB,), jnp.float32),
          pltpu.VMEM((_SB,), jnp.float32),
          pltpu.VMEM((_SB,), jnp.int32),
          pltpu.VMEM((_NG, _G), jnp.int32),
          pltpu.VMEM((_G,), jnp.float32),
          pltpu.VMEM((_ZR, _C), jnp.float32),
          pltpu.VMEM((_SB,), jnp.float32),
          pltpu.VMEM_SHARED((_ROWS, _C), jnp.float32),
          pltpu.VMEM_SHARED((_ROWS,), jnp.float32),
      ],
  )(feats_t, xs, ys, ms)


_YG = 8


def _tc_mean_body(sums_ref, cnts_ref, out_ref):
  s = sums_ref[0]
  c = cnts_ref[0]
  mean = jnp.where(c > 0, s / jnp.maximum(c, 1.0), 0.0)
  for yy in range(_YG):
    out_ref[0, :, yy, :] = mean[yy].T


def _tc_mean(sums, cnts):
  sums4 = sums.reshape(_B, _NY, _NX, _C)
  cnts4 = cnts.reshape(_B, _NY, _NX, 1)
  return pl.pallas_call(
      _tc_mean_body,
      grid=(_B, _NY // _YG),
      in_specs=[
          pl.BlockSpec((1, _YG, _NX, _C), lambda b, y: (b, y, 0, 0)),
          pl.BlockSpec((1, _YG, _NX, 1), lambda b, y: (b, y, 0, 0)),
      ],
      out_specs=pl.BlockSpec((1, _C, _YG, _NX), lambda b, y: (b, 0, y, 0)),
      out_shape=jax.ShapeDtypeStruct((_B, _C, _NY, _NX), jnp.float32),
  )(sums4, cnts4)


def kernel(fv_features, points_img, proj_masks):
  feats_t = jnp.transpose(fv_features.reshape(_B, _C, _N), (0, 2, 1))
  xs = points_img[:, 0].reshape(_B, _N)
  ys = points_img[:, 1].reshape(_B, _N)
  ms = proj_masks.reshape(_B, _N)
  sums, cnts = _sc_scatter(feats_t, xs, ys, ms)
  return _tc_mean(sums, cnts)
